# baseline (speedup 1.0000x reference)
import functools

import jax
import jax.numpy as jnp
from jax import lax
from jax.experimental import pallas as pl
from jax.experimental.pallas import tpu as pltpu
from jax.experimental.pallas import tpu_sc as plsc

_N_FIELDS = 26
_VOCAB = 100000
_EMB = 64
_NUM_DIM = 13
_BATCH = 16384
_TOK = _N_FIELDS + 1

_NC = 2
_NS = 16
_NW = _NC * _NS

_LANES = 16
_LPF = 100008
_PSCALE = 1

_ROWS = _BATCH * _N_FIELDS
_NB_W = _BATCH // _NW
_CB = 16
_NCHUNK = _NB_W // _CB
_QB = 4
_QROWS = _QB * _N_FIELDS
_NQ = _NB_W // _QB
_XPW = _NB_W * _N_FIELDS
_SROWS = _TOK * _EMB
_SCOLS = _CB + 1


def _num_matmul_t(x_num, W, b):
    bm = 2048

    def body(x_ref, w_ref, b_ref, o_ref):
        acc = (
            jnp.dot(x_ref[...], w_ref[...], preferred_element_type=jnp.float32)
            + b_ref[...]
        )
        o_ref[...] = acc.T

    return pl.pallas_call(
        body,
        out_shape=jax.ShapeDtypeStruct((_EMB, _BATCH), jnp.float32),
        grid=(_BATCH // bm,),
        in_specs=[
            pl.BlockSpec((bm, _NUM_DIM), lambda i: (i, 0)),
            pl.BlockSpec((_NUM_DIM, _EMB), lambda i: (0, 0)),
            pl.BlockSpec((1, _EMB), lambda i: (0, 0)),
        ],
        out_specs=pl.BlockSpec((_EMB, bm), lambda i: (0, i)),
    )(x_num, W, b.reshape(1, _EMB))


def _sc_body(xcat_hbm, numt_hbm, table_hbm, out_hbm,
             xbuf, sidx, gbuf, staging,
             gsem0, gsem1, osem0, osem1, nsem):
    wid = lax.axis_index("s") * _NC + lax.axis_index("c")
    iota = lax.iota(jnp.int32, _LANES)
    tv = table_hbm

    pltpu.sync_copy(xcat_hbm.at[pl.ds(wid * _XPW, _XPW)],
                    xbuf.at[pl.ds(0, _XPW)])

    gsems = (gsem0, gsem1)
    osems = (osem0, osem1)
    fconsts = [lax.rem(v * _LANES + iota, _N_FIELDS) for v in range(7)]

    def start_gather(gq, slot):
        base = gq * _QROWS
        for v in range(7):
            xv = xbuf[pl.ds(base + v * _LANES, _LANES)]
            sidx[slot, pl.ds(v * _LANES, _LANES)] = (
                xv * _PSCALE + fconsts[v] * (_PSCALE * (_VOCAB + 1)))
        pltpu.async_copy(
            tv.at[sidx.at[slot, pl.ds(0, _QROWS)]], gbuf.at[slot], gsems[slot]
        )

    def wait_gather(slot):
        pltpu.make_async_copy(
            tv.at[sidx.at[slot, pl.ds(0, _QROWS)]], gbuf.at[slot], gsems[slot]
        ).wait()

    def transpose_quarter(cc, q, slot):
        def tb(bl, carry):
            col = iota * 0 + (q * _QB + bl)
            for f in range(_N_FIELDS):
                line = bl * _N_FIELDS + f
                for eg in range(_EMB // _LANES):
                    v = gbuf[slot, line, pl.ds(eg * _LANES, _LANES)]
                    rows = iota + (f * _EMB + eg * _LANES)
                    plsc.store_scatter(staging.at[cc], [rows, col], v)
            return carry
        lax.fori_loop(0, _QB, tb, 0)

    def do_chunk(i, c, cc):
        b0 = wid * _NB_W + c * _CB

        @pl.when(i > 0)
        def _():
            pltpu.make_async_copy(
                staging.at[cc, :, pl.ds(0, _CB)],
                out_hbm.at[:, pl.ds(b0 - 2 * _CB, _CB)],
                osems[cc],
            ).wait()

        pltpu.async_copy(
            numt_hbm.at[:, pl.ds(b0, _CB)],
            staging.at[cc, pl.ds(_N_FIELDS * _EMB, _EMB), pl.ds(0, _CB)],
            nsem,
        )

        for q in range(_CB // _QB):
            gq = c * (_CB // _QB) + q

            @pl.when(gq + 1 < _NQ)
            def _():
                start_gather(gq + 1, (q + 1) % 2)

            wait_gather(q % 2)
            transpose_quarter(cc, q, q % 2)

        pltpu.make_async_copy(
            numt_hbm.at[:, pl.ds(b0, _CB)],
            staging.at[cc, pl.ds(_N_FIELDS * _EMB, _EMB), pl.ds(0, _CB)],
            nsem,
        ).wait()
        pltpu.async_copy(
            staging.at[cc, :, pl.ds(0, _CB)],
            out_hbm.at[:, pl.ds(b0, _CB)],
            osems[cc],
        )

    start_gather(0, 0)

    def pair(i, carry):
        do_chunk(i, 2 * i, 0)
        do_chunk(i, 2 * i + 1, 1)
        return carry

    lax.fori_loop(0, _NCHUNK // 2, pair, 0)

    for cc in range(2):
        b0 = wid * _NB_W + (_NCHUNK - 2 + cc) * _CB
        pltpu.make_async_copy(
            staging.at[cc, :, pl.ds(0, _CB)],
            out_hbm.at[:, pl.ds(b0, _CB)],
            osems[cc],
        ).wait()


_sc_gather = functools.partial(
    pl.kernel,
    out_type=jax.ShapeDtypeStruct((_SROWS, _BATCH), jnp.float32),
    mesh=plsc.VectorSubcoreMesh(core_axis_name="c", subcore_axis_name="s"),
    compiler_params=pltpu.CompilerParams(
        disable_bounds_checks=True, use_tc_tiling_on_sc=False,
        needs_layout_passes=False),
    scratch_types=[
        pltpu.VMEM((_XPW + _LANES,), jnp.int32),
        pltpu.VMEM((2, 7 * _LANES), jnp.int32),
        pltpu.VMEM((2, _QROWS, _EMB), jnp.float32),
        pltpu.VMEM((2, _SROWS, _SCOLS), jnp.float32),
        pltpu.SemaphoreType.DMA,
        pltpu.SemaphoreType.DMA,
        pltpu.SemaphoreType.DMA,
        pltpu.SemaphoreType.DMA,
        pltpu.SemaphoreType.DMA,
    ],
)(_sc_body)


def kernel(x_cat, x_num, tables, W, b):
    numt = _num_matmul_t(x_num, W, b)
    xflat = x_cat.astype(jnp.int32).reshape(_ROWS)
    tflat = tables.reshape(_N_FIELDS * (_VOCAB + 1), _EMB)
    out_t = _sc_gather(xflat, numt, tflat)
    return out_t.reshape(_TOK, _EMB, _BATCH).transpose(2, 0, 1)


# speedup vs baseline: 1.0008x; 1.0008x over previous
import functools

import jax
import jax.numpy as jnp
from jax import lax
from jax.experimental import pallas as pl
from jax.experimental.pallas import tpu as pltpu
from jax.experimental.pallas import tpu_sc as plsc

_N_FIELDS = 26
_VOCAB = 100000
_EMB = 64
_NUM_DIM = 13
_BATCH = 16384
_TOK = _N_FIELDS + 1

_NC = 2
_NS = 16
_NW = _NC * _NS

_LANES = 16
_LPF = 100008
_PSCALE = 1

_ROWS = _BATCH * _N_FIELDS
_NB_W = _BATCH // _NW
_CB = 16
_NCHUNK = _NB_W // _CB
_QB = 4
_QROWS = _QB * _N_FIELDS
_NQ = _NB_W // _QB
_XPW = _NB_W * _N_FIELDS
_SROWS = _TOK * _EMB
_SCOLS = _CB + 1


def _num_matmul_t(x_num, W, b):
    bm = 2048

    def body(x_ref, w_ref, b_ref, o_ref):
        acc = (
            jnp.dot(x_ref[...], w_ref[...], preferred_element_type=jnp.float32)
            + b_ref[...]
        )
        o_ref[...] = acc.T

    return pl.pallas_call(
        body,
        out_shape=jax.ShapeDtypeStruct((_EMB, _BATCH), jnp.float32),
        grid=(_BATCH // bm,),
        in_specs=[
            pl.BlockSpec((bm, _NUM_DIM), lambda i: (i, 0)),
            pl.BlockSpec((_NUM_DIM, _EMB), lambda i: (0, 0)),
            pl.BlockSpec((1, _EMB), lambda i: (0, 0)),
        ],
        out_specs=pl.BlockSpec((_EMB, bm), lambda i: (0, i)),
    )(x_num, W, b.reshape(1, _EMB))


def _sc_body(xcat_hbm, numt_hbm, table_hbm, out_hbm,
             xbuf, sidx, gbuf, staging,
             gsem0, gsem1, osem0, osem1, nsem):
    wid = lax.axis_index("s") * _NC + lax.axis_index("c")
    iota = lax.iota(jnp.int32, _LANES)

    pltpu.sync_copy(xcat_hbm.at[pl.ds(wid * _XPW, _XPW)],
                    xbuf.at[pl.ds(0, _XPW)])

    gsems = (gsem0, gsem1)
    osems = (osem0, osem1)
    fconsts = [lax.rem(v * _LANES + iota, _N_FIELDS) for v in range(7)]

    tv = table_hbm

    def start_gather(gq, slot):
        base = gq * _QROWS
        for v in range(7):
            xv = xbuf[pl.ds(base + v * _LANES, _LANES)]
            sidx[slot, pl.ds(v * _LANES, _LANES)] = (
                xv * _PSCALE + fconsts[v] * (_PSCALE * (_VOCAB + 1)))
        pltpu.async_copy(
            tv.at[sidx.at[slot, pl.ds(0, _QROWS)]], gbuf.at[slot], gsems[slot]
        )

    def wait_gather(slot):
        pltpu.make_async_copy(
            tv.at[sidx.at[slot, pl.ds(0, _QROWS)]], gbuf.at[slot], gsems[slot]
        ).wait()

    def transpose_quarter(cc, q, slot):
        def tb(bl, carry):
            col = iota * 0 + (q * _QB + bl)
            for f in range(_N_FIELDS):
                line = bl * _N_FIELDS + f
                for eg in range(_EMB // _LANES):
                    v = gbuf[slot, line, pl.ds(eg * _LANES, _LANES)]
                    rows = iota + (f * _EMB + eg * _LANES)
                    plsc.store_scatter(staging.at[cc], [rows, col], v)
            return carry
        lax.fori_loop(0, _QB, tb, 0)

    def do_chunk(i, c, cc):
        b0 = wid * _NB_W + c * _CB

        @pl.when(i > 0)
        def _():
            pltpu.make_async_copy(
                staging.at[cc, :, pl.ds(0, _CB)],
                out_hbm.at[:, pl.ds(b0 - 2 * _CB, _CB)],
                osems[cc],
            ).wait()

        pltpu.async_copy(
            numt_hbm.at[:, pl.ds(b0, _CB)],
            staging.at[cc, pl.ds(_N_FIELDS * _EMB, _EMB), pl.ds(0, _CB)],
            nsem,
        )

        for q in range(_CB // _QB):
            gq = c * (_CB // _QB) + q

            @pl.when(gq + 1 < _NQ)
            def _():
                start_gather(gq + 1, (q + 1) % 2)

            wait_gather(q % 2)
            transpose_quarter(cc, q, q % 2)

        pltpu.make_async_copy(
            numt_hbm.at[:, pl.ds(b0, _CB)],
            staging.at[cc, pl.ds(_N_FIELDS * _EMB, _EMB), pl.ds(0, _CB)],
            nsem,
        ).wait()
        pltpu.async_copy(
            staging.at[cc, :, pl.ds(0, _CB)],
            out_hbm.at[:, pl.ds(b0, _CB)],
            osems[cc],
        )

    start_gather(0, 0)

    def pair(i, carry):
        do_chunk(i, 2 * i, 0)
        do_chunk(i, 2 * i + 1, 1)
        return carry

    lax.fori_loop(0, _NCHUNK // 2, pair, 0)

    for cc in range(2):
        b0 = wid * _NB_W + (_NCHUNK - 2 + cc) * _CB
        pltpu.make_async_copy(
            staging.at[cc, :, pl.ds(0, _CB)],
            out_hbm.at[:, pl.ds(b0, _CB)],
            osems[cc],
        ).wait()


_sc_gather = functools.partial(
    pl.kernel,
    out_type=jax.ShapeDtypeStruct((_SROWS, _BATCH), jnp.float32),
    mesh=plsc.VectorSubcoreMesh(core_axis_name="c", subcore_axis_name="s"),
    compiler_params=pltpu.CompilerParams(
        disable_bounds_checks=True, use_tc_tiling_on_sc=False,
        needs_layout_passes=False),
    scratch_types=[
        pltpu.VMEM((_XPW + _LANES,), jnp.int32),
        pltpu.VMEM((2, 7 * _LANES), jnp.int32),
        pltpu.VMEM((2, _QROWS, _EMB), jnp.float32),
        pltpu.VMEM((2, _SROWS, _SCOLS), jnp.float32),
        pltpu.SemaphoreType.DMA,
        pltpu.SemaphoreType.DMA,
        pltpu.SemaphoreType.DMA,
        pltpu.SemaphoreType.DMA,
        pltpu.SemaphoreType.DMA,
    ],
)(_sc_body)


def kernel(x_cat, x_num, tables, W, b):
    numt = _num_matmul_t(x_num, W, b)
    xflat = x_cat.astype(jnp.int32).reshape(_ROWS)
    tflat = tables.reshape(_N_FIELDS * (_VOCAB + 1), _EMB)
    out_t = _sc_gather(xflat, numt, tflat)
    return out_t.reshape(_TOK, _EMB, _BATCH).transpose(2, 0, 1)


# speedup vs baseline: 4.6051x; 4.6013x over previous
import functools

import jax
import jax.numpy as jnp
from jax import lax
from jax.experimental import pallas as pl
from jax.experimental.pallas import tpu as pltpu
from jax.experimental.pallas import tpu_sc as plsc

_N_FIELDS = 26
_VOCAB = 100000
_EMB = 64
_NUM_DIM = 13
_BATCH = 16384
_TOK = _N_FIELDS + 1

_NC = 2
_NS = 16
_NW = _NC * _NS

_LANES = 16
_SUB = 8

_ROWS = _BATCH * _N_FIELDS
_NB_W = _BATCH // _NW
_QB = 1
_QROWS = _QB * _N_FIELDS
_OB = 4
_GPB = _OB // _QB
_NBLK = _NB_W // _OB
_NGRP = _NB_W // _QB
_XPW = _NB_W * _N_FIELDS


def _num_matmul(x_num, W, b):
    bm = 2048

    def body(x_ref, w_ref, b_ref, o_ref):
        acc = (
            jnp.dot(x_ref[...], w_ref[...], preferred_element_type=jnp.float32)
            + b_ref[...]
        )
        o_ref[:, 0:_EMB] = acc

    return pl.pallas_call(
        body,
        out_shape=jax.ShapeDtypeStruct((_BATCH, 2 * _EMB), jnp.float32),
        grid=(_BATCH // bm,),
        in_specs=[
            pl.BlockSpec((bm, _NUM_DIM), lambda i: (i, 0)),
            pl.BlockSpec((_NUM_DIM, _EMB), lambda i: (0, 0)),
            pl.BlockSpec((1, _EMB), lambda i: (0, 0)),
        ],
        out_specs=pl.BlockSpec((bm, 2 * _EMB), lambda i: (i, 0)),
    )(x_num, W, b.reshape(1, _EMB))


def _sc_body(xcat_hbm, num_hbm, table_hbm, out_hbm,
             xbuf, gbuf8, cbuf, numbuf,
             gsem0, gsem1, osem0, osem1, nsem):
    wid = lax.axis_index("s") * _NC + lax.axis_index("c")

    pltpu.sync_copy(xcat_hbm.at[pl.ds(wid * _XPW, _XPW)],
                    xbuf.at[pl.ds(0, _XPW)])

    gsems = (gsem0, gsem1)
    osems = (osem0, osem1)

    def start_group(g, slot):
        base = g * _QROWS

        def field(f, carry):
            for bl in range(_QB):
                r = bl * _N_FIELDS + f
                v = xbuf[pl.ds(base + r, _LANES)][0]
                v8 = pl.multiple_of((v // _SUB) * _SUB, _SUB)
                pltpu.async_copy(
                    table_hbm.at[f, pl.ds(v8, _SUB), :],
                    gbuf8.at[slot, r],
                    gsems[slot],
                )
            return carry

        lax.fori_loop(0, _N_FIELDS, field, 0)

    def wait_group(slot):
        pltpu.make_async_copy(
            table_hbm.at[0, pl.ds(0, _SUB * _QROWS), :].reshape(
                _QROWS, _SUB, _EMB),
            gbuf8.at[slot],
            gsems[slot],
        ).wait()

    def compact(g, slot, ib, q):
        base = g * _QROWS

        def field(f, carry):
            for bl in range(_QB):
                r = bl * _N_FIELDS + f
                voff = lax.rem(xbuf[pl.ds(base + r, _LANES)][0], _SUB)
                for eg in range(_EMB // _LANES):
                    cbuf[ib, q * _QB + bl, f, pl.ds(eg * _LANES, _LANES)] = (
                        gbuf8[slot, r, voff, pl.ds(eg * _LANES, _LANES)]
                    )
            return carry

        lax.fori_loop(0, _N_FIELDS, field, 0)

    def do_block(i, ib):
        blk = 2 * i + ib
        b0 = wid * _NB_W + blk * _OB

        @pl.when(i > 0)
        def _():
            pltpu.make_async_copy(
                cbuf.at[ib], out_hbm.at[pl.ds(b0 - 2 * _OB, _OB)], osems[ib]
            ).wait()

        pltpu.async_copy(
            num_hbm.at[pl.ds(b0, _OB), :], numbuf.at[ib], nsem)

        for q in range(_GPB):
            g = blk * _GPB + q
            slot = q % 2

            @pl.when(g + 1 < _NGRP)
            def _():
                start_group(g + 1, (q + 1) % 2)

            wait_group(slot)
            compact(g, slot, ib, q)

        pltpu.make_async_copy(
            num_hbm.at[pl.ds(b0, _OB), :], numbuf.at[ib], nsem).wait()
        for bl in range(_OB):
            for eg in range(_EMB // _LANES):
                cbuf[ib, bl, _N_FIELDS, pl.ds(eg * _LANES, _LANES)] = (
                    numbuf[ib, bl, pl.ds(eg * _LANES, _LANES)]
                )
        pltpu.async_copy(
            cbuf.at[ib], out_hbm.at[pl.ds(b0, _OB)], osems[ib])

    start_group(0, 0)

    def pair(i, carry):
        do_block(i, 0)
        do_block(i, 1)
        return carry

    lax.fori_loop(0, _NBLK // 2, pair, 0)

    for ib in range(2):
        b0 = wid * _NB_W + (_NBLK - 2 + ib) * _OB
        pltpu.make_async_copy(
            cbuf.at[ib], out_hbm.at[pl.ds(b0, _OB)], osems[ib]
        ).wait()


_sc_gather = functools.partial(
    pl.kernel,
    out_type=jax.ShapeDtypeStruct((_BATCH, _TOK, _EMB), jnp.float32),
    mesh=plsc.VectorSubcoreMesh(core_axis_name="c", subcore_axis_name="s"),
    scratch_types=[
        pltpu.VMEM((_XPW + _LANES,), jnp.int32),
        pltpu.VMEM((2, _QROWS, _SUB, _EMB), jnp.float32),
        pltpu.VMEM((2, _OB, _TOK, _EMB), jnp.float32),
        pltpu.VMEM((2, _OB, 2 * _EMB), jnp.float32),
        pltpu.SemaphoreType.DMA,
        pltpu.SemaphoreType.DMA,
        pltpu.SemaphoreType.DMA,
        pltpu.SemaphoreType.DMA,
        pltpu.SemaphoreType.DMA,
    ],
)(_sc_body)


def kernel(x_cat, x_num, tables, W, b):
    num = _num_matmul(x_num, W, b)
    xflat = x_cat.astype(jnp.int32).reshape(_ROWS)
    return _sc_gather(xflat, num, tables)
